# baseline (speedup 1.0000x reference)
import functools

import jax
import jax.numpy as jnp
from jax import lax
import numpy as np
from jax.experimental import pallas as pl
from jax.experimental.pallas import tpu as pltpu
from jax.experimental.pallas import tpu_sc as plsc

F = 26
B = 16384
FD = 100000
NC, NS, L = 2, 16, 16
NW = NC * NS
BPW = B // NW
CHUNK = F * BPW
NJ = BPW // L
SPLITS = ((0, 13), (13, 13))


def kernel(x, W, bias):
    offsets = jnp.asarray(np.arange(F, dtype=np.int32) * FD)
    xt = jnp.swapaxes(x, 0, 1) + offsets[:, None]
    wf = jnp.pad(W, ((0, 960), (0, 0))).reshape(-1)
    b16 = jnp.broadcast_to(bias.astype(jnp.float32), (L,))

    mesh = plsc.VectorSubcoreMesh(core_axis_name="c", subcore_axis_name="s")

    @functools.partial(
        pl.kernel,
        mesh=mesh,
        out_type=jax.ShapeDtypeStruct((B,), jnp.float32),
        compiler_params=pltpu.CompilerParams(needs_layout_passes=False),
        scratch_types=[
            pltpu.VMEM((CHUNK,), jnp.int32),
            pltpu.VMEM((CHUNK,), jnp.float32),
            pltpu.VMEM((L,), jnp.float32),
            pltpu.VMEM((BPW,), jnp.float32),
            pltpu.SemaphoreType.DMA,
            pltpu.SemaphoreType.DMA,
        ],
    )
    def sc_kernel(x_hbm, w_hbm, b_hbm, out_hbm, idx_v, rows_v, bias_v, acc_v, sem, sem2):
        wid = lax.axis_index("s") * NC + lax.axis_index("c")
        b0 = wid * BPW
        idx_copies = [
            pltpu.async_copy(
                x_hbm.at[f, pl.ds(b0, BPW)], idx_v.at[pl.ds(f * BPW, BPW)], sem2
            )
            for f in range(F)
        ]
        pltpu.sync_copy(b_hbm, bias_v)
        for c in idx_copies:
            c.wait()

        bounds = (0, 7, 14, 20, F)
        copies = [
            pltpu.async_copy(
                w_hbm.at[idx_v.at[pl.ds(bounds[s] * BPW, (bounds[s + 1] - bounds[s]) * BPW)]],
                rows_v.at[pl.ds(bounds[s] * BPW, (bounds[s + 1] - bounds[s]) * BPW)],
                sem,
            )
            for s in range(4)
        ]
        for s in range(4):
            copies[s].wait()
            first, last = s == 0, s == 3
            f_lo, f_hi = bounds[s], bounds[s + 1]

            def accum(j, carry, f_lo=f_lo, f_hi=f_hi, first=first, last=last):
                a = bias_v[...] if first else acc_v[pl.ds(j * L, L)]
                for f in range(f_lo, f_hi):
                    a = a + rows_v[pl.ds(f * BPW + j * L, L)]
                if last:
                    a = 1.0 / (1.0 + jnp.exp(-a))
                acc_v[pl.ds(j * L, L)] = a
                return carry

            lax.fori_loop(0, NJ, accum, 0)

        pltpu.sync_copy(acc_v, out_hbm.at[pl.ds(wid * BPW, BPW)])

    return sc_kernel(xt, wf, b16)


# speedup vs baseline: 1.0223x; 1.0223x over previous
import functools

import jax
import jax.numpy as jnp
from jax import lax
from jax.experimental import pallas as pl
from jax.experimental.pallas import tpu as pltpu
from jax.experimental.pallas import tpu_sc as plsc

F = 26
B = 16384
FD = 100000
NC, NS, L = 2, 16, 16
NW = NC * NS
BPW = B // NW
CHUNK = F * BPW
NJ = BPW // L
BOUNDS = (0, 7, 14, 20, F)
NS_STREAMS = len(BOUNDS) - 1


def kernel(x, W, bias):
    xt = jnp.swapaxes(x, 0, 1)
    wf = jnp.pad(W, ((0, 960), (0, 0))).reshape(-1)
    b16 = jnp.broadcast_to(bias.astype(jnp.float32), (L,))

    mesh = plsc.VectorSubcoreMesh(core_axis_name="c", subcore_axis_name="s")

    @functools.partial(
        pl.kernel,
        mesh=mesh,
        out_type=jax.ShapeDtypeStruct((B,), jnp.float32),
        compiler_params=pltpu.CompilerParams(needs_layout_passes=False),
        scratch_types=[
            pltpu.VMEM((F, BPW), jnp.int32),
            pltpu.VMEM((CHUNK,), jnp.int32),
            pltpu.VMEM((CHUNK,), jnp.float32),
            pltpu.VMEM((L,), jnp.float32),
            pltpu.VMEM((BPW,), jnp.float32),
            pltpu.SemaphoreType.DMA,
        ],
    )
    def sc_kernel(x_hbm, w_hbm, b_hbm, out_hbm, xv, idx_v, rows_v, bias_v, acc_v, sem):
        wid = lax.axis_index("s") * NC + lax.axis_index("c")
        pltpu.sync_copy(x_hbm.at[:, pl.ds(wid * BPW, BPW)], xv)
        pltpu.sync_copy(b_hbm, bias_v)

        copies = []
        for s in range(NS_STREAMS):
            f_lo, f_hi = BOUNDS[s], BOUNDS[s + 1]

            def mk_idx(j, carry, f_lo=f_lo, f_hi=f_hi):
                for f in range(f_lo, f_hi):
                    idx_v[pl.ds(f * BPW + j * L, L)] = xv[f, pl.ds(j * L, L)] + f * FD
                return carry

            lax.fori_loop(0, NJ, mk_idx, 0)
            copies.append(
                pltpu.async_copy(
                    w_hbm.at[idx_v.at[pl.ds(f_lo * BPW, (f_hi - f_lo) * BPW)]],
                    rows_v.at[pl.ds(f_lo * BPW, (f_hi - f_lo) * BPW)],
                    sem,
                )
            )

        for s in range(NS_STREAMS):
            copies[s].wait()
            first, last = s == 0, s == NS_STREAMS - 1
            f_lo, f_hi = BOUNDS[s], BOUNDS[s + 1]

            def accum(j, carry, f_lo=f_lo, f_hi=f_hi, first=first, last=last):
                a = bias_v[...] if first else acc_v[pl.ds(j * L, L)]
                for f in range(f_lo, f_hi):
                    a = a + rows_v[pl.ds(f * BPW + j * L, L)]
                if last:
                    a = 1.0 / (1.0 + jnp.exp(-a))
                acc_v[pl.ds(j * L, L)] = a
                return carry

            lax.fori_loop(0, NJ, accum, 0)

        pltpu.sync_copy(acc_v, out_hbm.at[pl.ds(wid * BPW, BPW)])

    return sc_kernel(xt, wf, b16)


# speedup vs baseline: 1.0353x; 1.0127x over previous
import functools

import jax
import jax.numpy as jnp
from jax import lax
from jax.experimental import pallas as pl
from jax.experimental.pallas import tpu as pltpu
from jax.experimental.pallas import tpu_sc as plsc

F = 26
B = 16384
FD = 100000
NC, NS, L = 2, 16, 16
NW = NC * NS
BPW = B // NW
CHUNK = F * BPW
NJ = BPW // L
BOUNDS = (0, 3, 9, 17, F)
NS_STREAMS = len(BOUNDS) - 1


def kernel(x, W, bias):
    xt = jnp.swapaxes(x, 0, 1)
    wf = jnp.pad(W, ((0, 960), (0, 0))).reshape(-1)
    b16 = jnp.broadcast_to(bias.astype(jnp.float32), (L,))

    mesh = plsc.VectorSubcoreMesh(core_axis_name="c", subcore_axis_name="s")

    @functools.partial(
        pl.kernel,
        mesh=mesh,
        out_type=jax.ShapeDtypeStruct((B,), jnp.float32),
        compiler_params=pltpu.CompilerParams(needs_layout_passes=False),
        scratch_types=[
            pltpu.VMEM((F, BPW), jnp.int32),
            pltpu.VMEM((CHUNK,), jnp.int32),
            pltpu.VMEM((CHUNK,), jnp.float32),
            pltpu.VMEM((L,), jnp.float32),
            pltpu.VMEM((BPW,), jnp.float32),
            pltpu.SemaphoreType.DMA,
        ],
    )
    def sc_kernel(x_hbm, w_hbm, b_hbm, out_hbm, xv, idx_v, rows_v, bias_v, acc_v, sem):
        wid = lax.axis_index("s") * NC + lax.axis_index("c")
        pltpu.sync_copy(x_hbm.at[:, pl.ds(wid * BPW, BPW)], xv)
        pltpu.sync_copy(b_hbm, bias_v)

        copies = []
        for s in range(NS_STREAMS):
            f_lo, f_hi = BOUNDS[s], BOUNDS[s + 1]

            def mk_idx(j, carry, f_lo=f_lo, f_hi=f_hi):
                for f in range(f_lo, f_hi):
                    idx_v[pl.ds(f * BPW + j * L, L)] = xv[f, pl.ds(j * L, L)] + f * FD
                return carry

            lax.fori_loop(0, NJ, mk_idx, 0)
            copies.append(
                pltpu.async_copy(
                    w_hbm.at[idx_v.at[pl.ds(f_lo * BPW, (f_hi - f_lo) * BPW)]],
                    rows_v.at[pl.ds(f_lo * BPW, (f_hi - f_lo) * BPW)],
                    sem,
                )
            )

        for s in range(NS_STREAMS):
            copies[s].wait()
            first, last = s == 0, s == NS_STREAMS - 1
            f_lo, f_hi = BOUNDS[s], BOUNDS[s + 1]

            def accum(j, carry, f_lo=f_lo, f_hi=f_hi, first=first, last=last):
                a = bias_v[...] if first else acc_v[pl.ds(j * L, L)]
                for f in range(f_lo, f_hi):
                    a = a + rows_v[pl.ds(f * BPW + j * L, L)]
                if last:
                    a = 1.0 / (1.0 + jnp.exp(-a))
                acc_v[pl.ds(j * L, L)] = a
                return carry

            lax.fori_loop(0, NJ, accum, 0)

        pltpu.sync_copy(acc_v, out_hbm.at[pl.ds(wid * BPW, BPW)])

    return sc_kernel(xt, wf, b16)
